# baseline (speedup 1.0000x reference)
import functools

import jax
import jax.numpy as jnp
import numpy as np
from jax import lax
from jax.experimental import pallas as pl
from jax.experimental.pallas import tpu as pltpu
from jax.experimental.pallas import tpu_sc as plsc

_NUM_LEVELS = 16
_FPL = 4
_LOG2 = 20
_BASE = 8
_FINEST = 512
_TEMPORAL = 32

_growth = np.exp((np.log(_FINEST) - np.log(_BASE)) / (_NUM_LEVELS - 1))
_SPATIAL = [int(np.floor(_BASE * _growth ** l)) for l in range(_NUM_LEVELS)]
_TEMP = [min(_TEMPORAL, s) for s in _SPATIAL]
_SIZES = [min(s ** 3 * t, 2 ** _LOG2) for s, t in zip(_SPATIAL, _TEMP)]
_OFFSETS = np.concatenate([[0], np.cumsum(_SIZES)]).astype(np.int64)
_TOTAL_ROWS = int(_OFFSETS[-1])

_MAIN_ROWS = (_TOTAL_ROWS // 128) * 128
_TAIL_ROWS = _TOTAL_ROWS - _MAIN_ROWS

_H1, _H2, _H3, _H4 = 73856093, 19349663, 83492791, 50331653


_RB = 32


def _hash_block(c_ref, idx_ref):
    x = c_ref[:, 0, :]
    y = c_ref[:, 1, :]
    z = c_ref[:, 2, :]
    t = c_ref[:, 3, :]
    for l in range(_NUM_LEVELS):
        sp = np.float32(_SPATIAL[l])
        st = np.float32(_TEMP[l])
        gx = jnp.floor(x * sp).astype(jnp.int32)
        gy = jnp.floor(y * sp).astype(jnp.int32)
        gz = jnp.floor(z * sp).astype(jnp.int32)
        gt = jnp.floor(t * st).astype(jnp.int32)
        h = (gx * _H1) ^ (gy * _H2) ^ (gz * _H3) ^ (gt * _H4)
        h = jnp.abs(h)
        if _SIZES[l] == 2 ** _LOG2:
            h = h & (2 ** _LOG2 - 1)
        else:
            h = jnp.mod(h, np.int32(_SIZES[l]))
        row = h + np.int32(_OFFSETS[l])
        phys = row + (row & np.int32(-128)) * 3
        if int(_OFFSETS[l]) + _SIZES[l] > _MAIN_ROWS:
            tail = row - np.int32(_MAIN_ROWS)
            for k in range(_FPL):
                enc = jnp.where(
                    row < np.int32(_MAIN_ROWS),
                    phys + np.int32(k * 128),
                    -(tail + np.int32(k * 128 + 1)),
                )
                idx_ref[:, l * _FPL + k, :] = enc
        else:
            for k in range(_FPL):
                idx_ref[:, l * _FPL + k, :] = phys + np.int32(k * 128)


def _hash_indices(coords_p):
    bsh = coords_p.shape[0]
    grid = (bsh // _RB,)
    return pl.pallas_call(
        _hash_block,
        grid=grid,
        in_specs=[pl.BlockSpec((_RB, 4, 128), lambda i: (i, 0, 0))],
        out_specs=pl.BlockSpec(
            (_RB, _NUM_LEVELS * _FPL, 128), lambda i: (i, 0, 0)),
        out_shape=jax.ShapeDtypeStruct(
            (bsh, _NUM_LEVELS * _FPL, 128), jnp.int32),
    )(coords_p)


_CHUNK_ROWS = 128
_L15_ROWS = [60 + 64 * j for j in range(_CHUNK_ROWS // 64)]

_CACHE_WORDS = ((int(_OFFSETS[2]) + 127) // 128) * 512
_CACHED_ROWS = [l * _FPL + k + 64 * j
                for j in range(_CHUNK_ROWS // 64)
                for l in (0, 1) for k in range(_FPL)]
_STREAM_RUNS = [(8, 56), (72, 56)]


def _make_sc_gather(rows_total):
    info = plsc.get_sparse_core_info()
    nc, ns = info.num_cores, info.num_subcores
    nw = nc * ns
    rows_per_w = rows_total // nw
    n_pairs = rows_per_w // (2 * _CHUNK_ROWS)
    mesh = plsc.VectorSubcoreMesh(core_axis_name="c", subcore_axis_name="s")

    @functools.partial(
        pl.kernel,
        mesh=mesh,
        out_type=jax.ShapeDtypeStruct((rows_total * 128,), jnp.float32),
        compiler_params=pltpu.CompilerParams(needs_layout_passes=False),
        scratch_types=[
            pltpu.VMEM((_CHUNK_ROWS * 128,), jnp.int32),
            pltpu.VMEM((_CHUNK_ROWS * 128,), jnp.int32),
            pltpu.VMEM((_CHUNK_ROWS * 128,), jnp.float32),
            pltpu.VMEM((_CHUNK_ROWS * 128,), jnp.float32),
            pltpu.VMEM((512,), jnp.float32),
            pltpu.VMEM((16, 128), jnp.int32),
            pltpu.VMEM((_CACHE_WORDS,), jnp.float32),
            pltpu.SemaphoreType.DMA,
            pltpu.SemaphoreType.DMA,
            pltpu.SemaphoreType.DMA,
            pltpu.SemaphoreType.DMA,
            pltpu.SemaphoreType.DMA,
            pltpu.SemaphoreType.DMA,
        ],
    )
    def sc_gather(idx_hbm, main_hbm, tail_hbm, out_hbm,
                  idx_v0, idx_v1, out_v0, out_v1, tail_v, orig_v, cache_v,
                  sem_i0, sem_i1, sem_g0, sem_g1, sem_o0, sem_o1):
        wid = lax.axis_index("s") * nc + lax.axis_index("c")
        cw = _CHUNK_ROWS * 128
        w_base = wid * rows_per_w * 128
        pltpu.async_copy(tail_hbm, tail_v, sem_i0).wait()
        pltpu.async_copy(
            main_hbm.at[pl.ds(0, _CACHE_WORDS)], cache_v, sem_i0).wait()

        def save_clamp(idx_v, obase):
            for li, r0 in enumerate(_L15_ROWS):
                for j in range(4):
                    rr = r0 + j
                    oi = obase + li * 4 + j
                    for v in range(8):
                        sl = pl.ds(rr * 128 + v * 16, 16)
                        ol = pl.ds(v * 16, 16)
                        orig = idx_v[sl]
                        orig_v[oi, ol] = orig
                        idx_v[sl] = jnp.maximum(orig, 0)

        def fire(idx_v, out_v, sem_g):
            return [pltpu.async_copy(
                main_hbm.at[idx_v.at[pl.ds(lo * 128, nrows * 128)]],
                out_v.at[pl.ds(lo * 128, nrows * 128)],
                sem_g,
            ) for lo, nrows in _STREAM_RUNS]

        def cached_gather(idx_v, out_v):
            for rr in _CACHED_ROWS:
                for v in range(8):
                    sl = pl.ds(rr * 128 + v * 16, 16)
                    out_v[sl] = plsc.load_gather(cache_v, [idx_v[sl]])

        def fixup(out_v, obase):
            for li, r0 in enumerate(_L15_ROWS):
                for j in range(4):
                    rr = r0 + j
                    oi = obase + li * 4 + j
                    for v in range(8):
                        sl = pl.ds(rr * 128 + v * 16, 16)
                        ol = pl.ds(v * 16, 16)
                        orig = orig_v[oi, ol]
                        m = orig < 0
                        toff = -orig - 1
                        patched = plsc.load_gather(tail_v, [toff], mask=m)
                        out_v[sl] = jnp.where(m, patched, out_v[sl])

        pltpu.async_copy(idx_hbm.at[pl.ds(w_base, cw)], idx_v0, sem_i0)
        pltpu.async_copy(idx_hbm.at[pl.ds(w_base + cw, cw)], idx_v1, sem_i1)

        def body(i, carry):
            base0 = w_base + (2 * i) * cw
            base1 = base0 + cw
            pltpu.make_async_copy(
                idx_hbm.at[pl.ds(base0, cw)], idx_v0, sem_i0).wait()
            save_clamp(idx_v0, 0)

            @pl.when(i > 0)
            def _():
                pltpu.make_async_copy(
                    out_v0, out_hbm.at[pl.ds(w_base, cw)], sem_o0).wait()

            c0 = fire(idx_v0, out_v0, sem_g0)
            cached_gather(idx_v0, out_v0)
            pltpu.make_async_copy(
                idx_hbm.at[pl.ds(base1, cw)], idx_v1, sem_i1).wait()
            save_clamp(idx_v1, 8)

            @pl.when(i > 0)
            def _():
                pltpu.make_async_copy(
                    out_v1, out_hbm.at[pl.ds(w_base, cw)], sem_o1).wait()

            c1 = fire(idx_v1, out_v1, sem_g1)
            cached_gather(idx_v1, out_v1)
            for c in c0:
                c.wait()
            fixup(out_v0, 0)
            pltpu.async_copy(out_v0, out_hbm.at[pl.ds(base0, cw)], sem_o0)

            @pl.when(i < n_pairs - 1)
            def _():
                pltpu.async_copy(
                    idx_hbm.at[pl.ds(base0 + 2 * cw, cw)], idx_v0, sem_i0)

            for c in c1:
                c.wait()
            fixup(out_v1, 8)
            pltpu.async_copy(out_v1, out_hbm.at[pl.ds(base1, cw)], sem_o1)

            @pl.when(i < n_pairs - 1)
            def _():
                pltpu.async_copy(
                    idx_hbm.at[pl.ds(base1 + 2 * cw, cw)], idx_v1, sem_i1)

            return carry

        lax.fori_loop(0, n_pairs, body, 0)
        pltpu.make_async_copy(
            out_v0, out_hbm.at[pl.ds(w_base, cw)], sem_o0).wait()
        pltpu.make_async_copy(
            out_v1, out_hbm.at[pl.ds(w_base, cw)], sem_o1).wait()

    return sc_gather


_PIPE = 4


def kernel(coordinates, tables):
    b, s, h, w, _ = coordinates.shape
    bsh = b * s * h
    coords_p = coordinates.transpose(0, 1, 2, 4, 3).reshape(bsh, 4, w)
    nblk = _MAIN_ROWS // 128
    main_flat = (tables[:_MAIN_ROWS].T
                 .reshape(4, nblk, 128)
                 .transpose(1, 0, 2)
                 .reshape(_MAIN_ROWS * 4))
    tail = tables[_MAIN_ROWS:]
    tail_pad = jnp.zeros((128, 4), jnp.float32).at[:_TAIL_ROWS].set(tail)
    tail_flat = tail_pad.T.reshape(512)

    bsh_slice = bsh // _PIPE
    rows_slice = bsh_slice * _NUM_LEVELS * _FPL
    sc = _make_sc_gather(rows_slice)
    outs = []
    for p in range(_PIPE):
        cp = lax.slice_in_dim(coords_p, p * bsh_slice, (p + 1) * bsh_slice)
        idx_p = _hash_indices(cp)
        idx1d = idx_p.reshape(rows_slice * w)
        outs.append(sc(idx1d, main_flat, tail_flat))
    out = jnp.concatenate(outs, axis=0)
    out5 = out.reshape(b, s, h, _NUM_LEVELS * _FPL, w)
    return out5.transpose(0, 1, 2, 4, 3)


# speedup vs baseline: 1.0148x; 1.0148x over previous
import functools

import jax
import jax.numpy as jnp
import numpy as np
from jax import lax
from jax.experimental import pallas as pl
from jax.experimental.pallas import tpu as pltpu
from jax.experimental.pallas import tpu_sc as plsc

_NUM_LEVELS = 16
_FPL = 4
_LOG2 = 20
_BASE = 8
_FINEST = 512
_TEMPORAL = 32

_growth = np.exp((np.log(_FINEST) - np.log(_BASE)) / (_NUM_LEVELS - 1))
_SPATIAL = [int(np.floor(_BASE * _growth ** l)) for l in range(_NUM_LEVELS)]
_TEMP = [min(_TEMPORAL, s) for s in _SPATIAL]
_SIZES = [min(s ** 3 * t, 2 ** _LOG2) for s, t in zip(_SPATIAL, _TEMP)]
_OFFSETS = np.concatenate([[0], np.cumsum(_SIZES)]).astype(np.int64)
_TOTAL_ROWS = int(_OFFSETS[-1])

_MAIN_ROWS = (_TOTAL_ROWS // 128) * 128
_TAIL_ROWS = _TOTAL_ROWS - _MAIN_ROWS

_H1, _H2, _H3, _H4 = 73856093, 19349663, 83492791, 50331653


_RB = 32


def _hash_block(c_ref, idx_ref):
    x = c_ref[:, 0, :]
    y = c_ref[:, 1, :]
    z = c_ref[:, 2, :]
    t = c_ref[:, 3, :]
    for l in range(_NUM_LEVELS):
        sp = np.float32(_SPATIAL[l])
        st = np.float32(_TEMP[l])
        gx = jnp.floor(x * sp).astype(jnp.int32)
        gy = jnp.floor(y * sp).astype(jnp.int32)
        gz = jnp.floor(z * sp).astype(jnp.int32)
        gt = jnp.floor(t * st).astype(jnp.int32)
        h = (gx * _H1) ^ (gy * _H2) ^ (gz * _H3) ^ (gt * _H4)
        h = jnp.abs(h)
        if _SIZES[l] == 2 ** _LOG2:
            h = h & (2 ** _LOG2 - 1)
        else:
            h = jnp.mod(h, np.int32(_SIZES[l]))
        row = h + np.int32(_OFFSETS[l])
        phys = row + (row & np.int32(-128)) * 3
        if int(_OFFSETS[l]) + _SIZES[l] > _MAIN_ROWS:
            tail = row - np.int32(_MAIN_ROWS)
            for k in range(_FPL):
                enc = jnp.where(
                    row < np.int32(_MAIN_ROWS),
                    phys + np.int32(k * 128),
                    -(tail + np.int32(k * 128 + 1)),
                )
                idx_ref[:, l * _FPL + k, :] = enc
        else:
            for k in range(_FPL):
                idx_ref[:, l * _FPL + k, :] = phys + np.int32(k * 128)


def _hash_indices(coords_p):
    bsh = coords_p.shape[0]
    grid = (bsh // _RB,)
    return pl.pallas_call(
        _hash_block,
        grid=grid,
        in_specs=[pl.BlockSpec((_RB, 4, 128), lambda i: (i, 0, 0))],
        out_specs=pl.BlockSpec(
            (_RB, _NUM_LEVELS * _FPL, 128), lambda i: (i, 0, 0)),
        out_shape=jax.ShapeDtypeStruct(
            (bsh, _NUM_LEVELS * _FPL, 128), jnp.int32),
    )(coords_p)


_CHUNK_ROWS = 128
_L15_ROWS = [60 + 64 * j for j in range(_CHUNK_ROWS // 64)]

_CACHE_WORDS = ((int(_OFFSETS[2]) + 127) // 128) * 512
_CACHED_ROWS = [l * _FPL + k + 64 * j
                for j in range(_CHUNK_ROWS // 64)
                for l in (0, 1) for k in range(_FPL)]
_STREAM_RUNS = [(8, 56), (72, 56)]


def _make_sc_gather(rows_total):
    info = plsc.get_sparse_core_info()
    nc, ns = info.num_cores, info.num_subcores
    nw = nc * ns
    rows_per_w = rows_total // nw
    n_pairs = rows_per_w // (2 * _CHUNK_ROWS)
    mesh = plsc.VectorSubcoreMesh(core_axis_name="c", subcore_axis_name="s")

    @functools.partial(
        pl.kernel,
        mesh=mesh,
        out_type=jax.ShapeDtypeStruct((rows_total * 128,), jnp.float32),
        compiler_params=pltpu.CompilerParams(needs_layout_passes=False),
        scratch_types=[
            pltpu.VMEM((_CHUNK_ROWS * 128,), jnp.int32),
            pltpu.VMEM((_CHUNK_ROWS * 128,), jnp.int32),
            pltpu.VMEM((_CHUNK_ROWS * 128,), jnp.float32),
            pltpu.VMEM((_CHUNK_ROWS * 128,), jnp.float32),
            pltpu.VMEM((512,), jnp.float32),
            pltpu.VMEM((16, 128), jnp.int32),
            pltpu.VMEM((_CACHE_WORDS,), jnp.float32),
            pltpu.SemaphoreType.DMA,
            pltpu.SemaphoreType.DMA,
            pltpu.SemaphoreType.DMA,
            pltpu.SemaphoreType.DMA,
            pltpu.SemaphoreType.DMA,
            pltpu.SemaphoreType.DMA,
        ],
    )
    def sc_gather(idx_hbm, main_hbm, tail_hbm, out_hbm,
                  idx_v0, idx_v1, out_v0, out_v1, tail_v, orig_v, cache_v,
                  sem_i0, sem_i1, sem_g0, sem_g1, sem_o0, sem_o1):
        wid = lax.axis_index("s") * nc + lax.axis_index("c")
        cw = _CHUNK_ROWS * 128
        w_base = wid * rows_per_w * 128
        pltpu.async_copy(tail_hbm, tail_v, sem_i0).wait()
        pltpu.async_copy(
            main_hbm.at[pl.ds(0, _CACHE_WORDS)], cache_v, sem_i0).wait()

        def save_clamp(idx_v, obase):
            for li, r0 in enumerate(_L15_ROWS):
                for j in range(4):
                    rr = r0 + j
                    oi = obase + li * 4 + j
                    for v in range(8):
                        sl = pl.ds(rr * 128 + v * 16, 16)
                        ol = pl.ds(v * 16, 16)
                        orig = idx_v[sl]
                        orig_v[oi, ol] = orig
                        idx_v[sl] = jnp.maximum(orig, 0)

        def fire(idx_v, out_v, sem_g):
            return [pltpu.async_copy(
                main_hbm.at[idx_v.at[pl.ds(lo * 128, nrows * 128)]],
                out_v.at[pl.ds(lo * 128, nrows * 128)],
                sem_g,
            ) for lo, nrows in _STREAM_RUNS]

        def cached_gather(idx_v, out_v):
            for rr in _CACHED_ROWS:
                for v in range(8):
                    sl = pl.ds(rr * 128 + v * 16, 16)
                    out_v[sl] = plsc.load_gather(cache_v, [idx_v[sl]])

        def fixup(out_v, obase):
            for li, r0 in enumerate(_L15_ROWS):
                for j in range(4):
                    rr = r0 + j
                    oi = obase + li * 4 + j
                    for v in range(8):
                        sl = pl.ds(rr * 128 + v * 16, 16)
                        ol = pl.ds(v * 16, 16)
                        orig = orig_v[oi, ol]
                        m = orig < 0
                        toff = -orig - 1
                        patched = plsc.load_gather(tail_v, [toff], mask=m)
                        out_v[sl] = jnp.where(m, patched, out_v[sl])

        pltpu.async_copy(idx_hbm.at[pl.ds(w_base, cw)], idx_v0, sem_i0)
        pltpu.async_copy(idx_hbm.at[pl.ds(w_base + cw, cw)], idx_v1, sem_i1)

        def body(i, carry):
            base0 = w_base + (2 * i) * cw
            base1 = base0 + cw
            pltpu.make_async_copy(
                idx_hbm.at[pl.ds(base0, cw)], idx_v0, sem_i0).wait()
            save_clamp(idx_v0, 0)

            @pl.when(i > 0)
            def _():
                pltpu.make_async_copy(
                    out_v0, out_hbm.at[pl.ds(w_base, cw)], sem_o0).wait()

            c0 = fire(idx_v0, out_v0, sem_g0)
            cached_gather(idx_v0, out_v0)
            pltpu.make_async_copy(
                idx_hbm.at[pl.ds(base1, cw)], idx_v1, sem_i1).wait()
            save_clamp(idx_v1, 8)

            @pl.when(i > 0)
            def _():
                pltpu.make_async_copy(
                    out_v1, out_hbm.at[pl.ds(w_base, cw)], sem_o1).wait()

            c1 = fire(idx_v1, out_v1, sem_g1)
            cached_gather(idx_v1, out_v1)
            for c in c0:
                c.wait()
            fixup(out_v0, 0)
            pltpu.async_copy(out_v0, out_hbm.at[pl.ds(base0, cw)], sem_o0)

            @pl.when(i < n_pairs - 1)
            def _():
                pltpu.async_copy(
                    idx_hbm.at[pl.ds(base0 + 2 * cw, cw)], idx_v0, sem_i0)

            for c in c1:
                c.wait()
            fixup(out_v1, 8)
            pltpu.async_copy(out_v1, out_hbm.at[pl.ds(base1, cw)], sem_o1)

            @pl.when(i < n_pairs - 1)
            def _():
                pltpu.async_copy(
                    idx_hbm.at[pl.ds(base1 + 2 * cw, cw)], idx_v1, sem_i1)

            return carry

        lax.fori_loop(0, n_pairs, body, 0)
        pltpu.make_async_copy(
            out_v0, out_hbm.at[pl.ds(w_base, cw)], sem_o0).wait()
        pltpu.make_async_copy(
            out_v1, out_hbm.at[pl.ds(w_base, cw)], sem_o1).wait()

    return sc_gather


_PIPE = 8


def kernel(coordinates, tables):
    b, s, h, w, _ = coordinates.shape
    bsh = b * s * h
    coords_p = coordinates.transpose(0, 1, 2, 4, 3).reshape(bsh, 4, w)
    nblk = _MAIN_ROWS // 128
    main_flat = (tables[:_MAIN_ROWS].T
                 .reshape(4, nblk, 128)
                 .transpose(1, 0, 2)
                 .reshape(_MAIN_ROWS * 4))
    tail = tables[_MAIN_ROWS:]
    tail_pad = jnp.zeros((128, 4), jnp.float32).at[:_TAIL_ROWS].set(tail)
    tail_flat = tail_pad.T.reshape(512)

    bsh_slice = bsh // _PIPE
    rows_slice = bsh_slice * _NUM_LEVELS * _FPL
    sc = _make_sc_gather(rows_slice)
    outs = []
    for p in range(_PIPE):
        cp = lax.slice_in_dim(coords_p, p * bsh_slice, (p + 1) * bsh_slice)
        idx_p = _hash_indices(cp)
        idx1d = idx_p.reshape(rows_slice * w)
        outs.append(sc(idx1d, main_flat, tail_flat))
    out = jnp.concatenate(outs, axis=0)
    out5 = out.reshape(b, s, h, _NUM_LEVELS * _FPL, w)
    return out5.transpose(0, 1, 2, 4, 3)
